# baseline (speedup 1.0000x reference)
import functools

import jax
import jax.numpy as jnp
from jax import lax
from jax.experimental import pallas as pl
from jax.experimental.pallas import tpu as pltpu
from jax.experimental.pallas import tpu_sc as plsc

N = 10000
E = 320000
D = 128
H = 128
G = 64
OUT = 1

NC = 2
NS = 16
NW = NC * NS
EPW = E // NW
K = 80
EPWP = 10240
EP = NW * EPWP
NCHUNK = EPWP // K
NRB = 4
NIB = 8
WCHUNK = 80
NWCHUNK = N // WCHUNK
WPT = -(-NWCHUNK // NS)

BLK = 1000
NBLK = N // BLK


def _segsum_sc(table, idx3, zeros):
  mesh = plsc.VectorSubcoreMesh(core_axis_name="c", subcore_axis_name="s")

  @functools.partial(
      pl.kernel,
      out_type=jax.ShapeDtypeStruct((NC * N, D), jnp.float32),
      mesh=mesh,
      scratch_types=[
          pltpu.VMEM((2 * NIB, K), jnp.int32),
          [pltpu.VMEM((K, D), jnp.float32)] * NRB,
          pltpu.VMEM_SHARED((N + 128, D), jnp.float32),
          [pltpu.SemaphoreType.DMA] * NIB,
          [pltpu.SemaphoreType.DMA] * NRB,
          [pltpu.SemaphoreType.DMA] * NRB,
      ],
  )
  def k(table_h, idx_h, zeros_h, out_h,
        idx_v, rows, acc_sh, isem, gsem, ssem):
    c = lax.axis_index("c")
    s = lax.axis_index("s")
    wid = s * NC + c
    base = wid * NCHUNK

    def idx_start(i, b):
      pltpu.async_copy(idx_h.at[base + i], idx_v.at[pl.ds(2 * b, 2)], isem[b])

    def idx_wait(i, b):
      pltpu.make_async_copy(idx_h.at[base + i], idx_v.at[pl.ds(2 * b, 2)],
                            isem[b]).wait()

    def gather_start(q, r):
      pltpu.async_copy(table_h.at[idx_v.at[2 * q]], rows[r], gsem[r])

    def gather_wait(q, r):
      pltpu.make_async_copy(table_h.at[idx_v.at[2 * q]], rows[r],
                            gsem[r]).wait()

    def scatter_start(q, r):
      pltpu.async_copy(rows[r], acc_sh.at[idx_v.at[2 * q + 1]], ssem[r],
                       add=True)

    def scatter_wait(q, r):
      pltpu.make_async_copy(rows[r], acc_sh.at[idx_v.at[2 * q + 1]],
                            ssem[r]).wait()

    def zbody(j, carry):
      m = s + j * NS

      @pl.when(m < NWCHUNK)
      def _():
        pltpu.sync_copy(zeros_h, acc_sh.at[pl.ds(m * WCHUNK, WCHUNK)])

      return carry

    lax.fori_loop(0, WPT, zbody, 0)
    plsc.subcore_barrier()

    for q in range(4):
      idx_start(q, q)
    idx_wait(0, 0)
    gather_start(0, 0)

    def body(t, carry):
      for u in range(NIB):
        i = t * NIB + u
        r2 = (u + 2) % NRB
        q2 = (u + 2) % NIB
        q4 = (u + 4) % NIB

        @pl.when(i >= 2)
        def _():
          scatter_wait(q2, r2)

        @pl.when(i + 4 < NCHUNK)
        def _():
          idx_start(i + 4, q4)

        @pl.when(i + 1 < NCHUNK)
        def _():
          idx_wait(i + 1, (u + 1) % NIB)
          gather_start((u + 1) % NIB, (u + 1) % NRB)

        gather_wait(u % NIB, u % NRB)
        scatter_start(u % NIB, u % NRB)
      return carry

    lax.fori_loop(0, NCHUNK // NIB, body, 0)

    for u in range(2):
      scatter_wait((NCHUNK - 2 + u) % NIB, (NCHUNK - 2 + u) % NRB)
    plsc.subcore_barrier()

    def wbody(j, carry):
      m = s + j * NS

      @pl.when(m < NWCHUNK)
      def _():
        r0 = m * WCHUNK
        pltpu.sync_copy(acc_sh.at[pl.ds(r0, WCHUNK)],
                        out_h.at[pl.ds(c * N + r0, WCHUNK)])

      return carry

    lax.fori_loop(0, WPT, wbody, 0)

  return k(table, idx3, zeros)


def _gconv_dense(parts, x, W_rel, W_root, b_rel):
  def body(p0_ref, p1_ref, x_ref, wr_ref, wt_ref, b_ref, o_ref):
    agg = p0_ref[...] + p1_ref[...]
    acc = jnp.dot(agg, wr_ref[...], preferred_element_type=jnp.float32)
    acc += jnp.dot(x_ref[...], wt_ref[...], preferred_element_type=jnp.float32)
    o_ref[...] = jnp.maximum(acc + b_ref[...], 0.0)

  return pl.pallas_call(
      body,
      grid=(NBLK,),
      in_specs=[
          pl.BlockSpec((BLK, D), lambda i: (i, 0)),
          pl.BlockSpec((BLK, D), lambda i: (i + NBLK, 0)),
          pl.BlockSpec((BLK, D), lambda i: (i, 0)),
          pl.BlockSpec((D, H), lambda i: (0, 0)),
          pl.BlockSpec((D, H), lambda i: (0, 0)),
          pl.BlockSpec((1, H), lambda i: (0, 0)),
      ],
      out_specs=pl.BlockSpec((BLK, H), lambda i: (i, 0)),
      out_shape=jax.ShapeDtypeStruct((N, H), jnp.float32),
  )(parts, parts, x, W_rel, W_root, b_rel.reshape(1, H))


def _final(parts, h1, W_rel, W_root, b_rel, batch3, W1, b1, gamma, beta,
           W2p, b2p):
  def body(p0_ref, p1_ref, h1_ref, wr_ref, wt_ref, br_ref, bt_ref,
           w1_ref, b1_ref, g_ref, be_ref, w2_ref, b2_ref, o_ref, pacc):
    i = pl.program_id(0)

    @pl.when(i == 0)
    def _():
      pacc[...] = jnp.zeros_like(pacc)

    agg = p0_ref[...] + p1_ref[...]
    acc = jnp.dot(agg, wr_ref[...], preferred_element_type=jnp.float32)
    acc += jnp.dot(h1_ref[...], wt_ref[...], preferred_element_type=jnp.float32)
    h2 = jnp.maximum(acc + br_ref[...], 0.0)

    b = bt_ref[0]
    gid = lax.broadcasted_iota(jnp.int32, (G, BLK), 0)
    onehot = (gid == b).astype(jnp.float32)
    pacc[...] += jnp.dot(onehot, h2, preferred_element_type=jnp.float32)

    @pl.when(i == NBLK - 1)
    def _():
      z = jnp.dot(pacc[...], w1_ref[...],
                  preferred_element_type=jnp.float32) + b1_ref[...]
      mu = jnp.mean(z, axis=0, keepdims=True)
      var = jnp.mean((z - mu) * (z - mu), axis=0, keepdims=True)
      zn = (z - mu) * lax.rsqrt(var + 1e-5) * g_ref[...] + be_ref[...]
      zn = jnp.maximum(zn, 0.0)
      logits = jnp.dot(zn, w2_ref[...],
                       preferred_element_type=jnp.float32) + b2_ref[...]
      o_ref[...] = jax.nn.sigmoid(logits)

  H4 = 4 * H
  return pl.pallas_call(
      body,
      grid=(NBLK,),
      in_specs=[
          pl.BlockSpec((BLK, D), lambda i: (i, 0)),
          pl.BlockSpec((BLK, D), lambda i: (i + NBLK, 0)),
          pl.BlockSpec((BLK, H), lambda i: (i, 0)),
          pl.BlockSpec((H, H), lambda i: (0, 0)),
          pl.BlockSpec((H, H), lambda i: (0, 0)),
          pl.BlockSpec((1, H), lambda i: (0, 0)),
          pl.BlockSpec((1, 1, BLK), lambda i: (i, 0, 0)),
          pl.BlockSpec((H, H4), lambda i: (0, 0)),
          pl.BlockSpec((1, H4), lambda i: (0, 0)),
          pl.BlockSpec((1, H4), lambda i: (0, 0)),
          pl.BlockSpec((1, H4), lambda i: (0, 0)),
          pl.BlockSpec((H4, 128), lambda i: (0, 0)),
          pl.BlockSpec((1, 128), lambda i: (0, 0)),
      ],
      out_specs=pl.BlockSpec((G, 128), lambda i: (0, 0)),
      out_shape=jax.ShapeDtypeStruct((G, 128), jnp.float32),
      scratch_shapes=[pltpu.VMEM((G, H), jnp.float32)],
  )(parts, parts, h1, W_rel, W_root, b_rel.reshape(1, H),
    batch3, W1, b1.reshape(1, H4), gamma.reshape(1, H4),
    beta.reshape(1, H4), W2p, b2p)


def kernel(x, edge_index, batch, W_rel0, b_rel0, W_root0, W_rel1, b_rel1,
           W_root1, W1, b1, gamma, beta, W2, b2):
  pad = EP - E
  src_p = jnp.concatenate([edge_index[0], jnp.zeros((pad,), jnp.int32)])
  dst_p = jnp.concatenate(
      [edge_index[1], N + (jnp.arange(pad, dtype=jnp.int32) % 128)])
  idx3 = jnp.stack([src_p.reshape(EP // K, K),
                    dst_p.reshape(EP // K, K)], axis=1)
  zeros = jnp.zeros((WCHUNK, D), jnp.float32)
  batch3 = batch.reshape(NBLK, 1, BLK)
  W2p = jnp.pad(W2, ((0, 0), (0, 128 - OUT)))
  b2p = jnp.pad(b2, (0, 128 - OUT)).reshape(1, 128)

  parts0 = _segsum_sc(x, idx3, zeros)
  h1 = _gconv_dense(parts0, x, W_rel0, W_root0, b_rel0)
  parts1 = _segsum_sc(h1, idx3, zeros)
  out = _final(parts1, h1, W_rel1, W_root1, b_rel1, batch3, W1, b1,
               gamma, beta, W2p, b2p)
  return out[:, :OUT]


# speedup vs baseline: 3.2720x; 3.2720x over previous
import functools

import jax
import jax.numpy as jnp
from jax import lax
from jax.experimental import pallas as pl
from jax.experimental.pallas import tpu as pltpu
from jax.experimental.pallas import tpu_sc as plsc

N = 10000
E = 320000
D = 128
H = 128
G = 64
OUT = 1

NC = 2
NS = 16
NW = NC * NS
EPW = E // NW
K = 80
NCHUNK = EPW // K
NRB = 4
NIB = 8
NITER = -(-NCHUNK // NIB) * NIB
WCHUNK = 80
NWCHUNK = N // WCHUNK
WPT = -(-NWCHUNK // NS)

BLK = 1000
NBLK = N // BLK


def _segsum_sc(table, idx3, zeros):
  mesh = plsc.VectorSubcoreMesh(core_axis_name="c", subcore_axis_name="s")

  @functools.partial(
      pl.kernel,
      out_type=jax.ShapeDtypeStruct((NC * N, D), jnp.float32),
      mesh=mesh,
      scratch_types=[
          pltpu.VMEM((2 * NIB, K), jnp.int32),
          [pltpu.VMEM((K, D), jnp.float32)] * NRB,
          pltpu.VMEM_SHARED((N, D), jnp.float32),
          [pltpu.SemaphoreType.DMA] * NIB,
          [pltpu.SemaphoreType.DMA] * NRB,
          [pltpu.SemaphoreType.DMA] * NRB,
      ],
  )
  def k(table_h, idx_h, zeros_h, out_h,
        idx_v, rows, acc_sh, isem, gsem, ssem):
    c = lax.axis_index("c")
    s = lax.axis_index("s")
    wid = s * NC + c
    base = wid * NCHUNK

    def idx_start(i, b):
      pltpu.async_copy(idx_h.at[base + i], idx_v.at[pl.ds(2 * b, 2)], isem[b])

    def idx_wait(i, b):
      pltpu.make_async_copy(idx_h.at[base + i], idx_v.at[pl.ds(2 * b, 2)],
                            isem[b]).wait()

    def gather_start(q, r):
      pltpu.async_copy(table_h.at[idx_v.at[2 * q]], rows[r], gsem[r])

    def gather_wait(q, r):
      pltpu.make_async_copy(table_h.at[idx_v.at[2 * q]], rows[r],
                            gsem[r]).wait()

    def scatter_start(q, r):
      pltpu.async_copy(rows[r], acc_sh.at[idx_v.at[2 * q + 1]], ssem[r],
                       add=True)

    def scatter_wait(q, r):
      pltpu.make_async_copy(rows[r], acc_sh.at[idx_v.at[2 * q + 1]],
                            ssem[r]).wait()

    def zbody(j, carry):
      m = s + j * NS

      @pl.when(m < NWCHUNK)
      def _():
        pltpu.sync_copy(zeros_h, acc_sh.at[pl.ds(m * WCHUNK, WCHUNK)])

      return carry

    lax.fori_loop(0, WPT, zbody, 0)
    plsc.subcore_barrier()

    for q in range(4):
      idx_start(q, q)
    for q in range(2):
      idx_wait(q, q)
      gather_start(q, q)

    def body(t, carry):
      for u in range(NIB):
        i = t * NIB + u
        r2 = (u + 2) % NRB
        q2 = (u + 2) % NIB
        q4 = (u + 4) % NIB

        @pl.when((i >= 2) & (i < NCHUNK + 2))
        def _():
          scatter_wait(q2, r2)

        @pl.when(i + 4 < NCHUNK)
        def _():
          idx_start(i + 4, q4)

        @pl.when(i + 2 < NCHUNK)
        def _():
          idx_wait(i + 2, q2)
          gather_start(q2, r2)

        @pl.when(i < NCHUNK)
        def _():
          gather_wait(u % NIB, u % NRB)
          scatter_start(u % NIB, u % NRB)
      return carry

    lax.fori_loop(0, NITER // NIB, body, 0)

    plsc.subcore_barrier()

    def wbody(j, carry):
      m = s + j * NS

      @pl.when(m < NWCHUNK)
      def _():
        r0 = m * WCHUNK
        pltpu.sync_copy(acc_sh.at[pl.ds(r0, WCHUNK)],
                        out_h.at[pl.ds(c * N + r0, WCHUNK)])

      return carry

    lax.fori_loop(0, WPT, wbody, 0)

  return k(table, idx3, zeros)


def _gconv_dense(parts, x, W_rel, W_root, b_rel):
  def body(p0_ref, p1_ref, x_ref, wr_ref, wt_ref, b_ref, o_ref):
    agg = p0_ref[...] + p1_ref[...]
    acc = jnp.dot(agg, wr_ref[...], preferred_element_type=jnp.float32)
    acc += jnp.dot(x_ref[...], wt_ref[...], preferred_element_type=jnp.float32)
    o_ref[...] = jnp.maximum(acc + b_ref[...], 0.0)

  return pl.pallas_call(
      body,
      grid=(NBLK,),
      in_specs=[
          pl.BlockSpec((BLK, D), lambda i: (i, 0)),
          pl.BlockSpec((BLK, D), lambda i: (i + NBLK, 0)),
          pl.BlockSpec((BLK, D), lambda i: (i, 0)),
          pl.BlockSpec((D, H), lambda i: (0, 0)),
          pl.BlockSpec((D, H), lambda i: (0, 0)),
          pl.BlockSpec((1, H), lambda i: (0, 0)),
      ],
      out_specs=pl.BlockSpec((BLK, H), lambda i: (i, 0)),
      out_shape=jax.ShapeDtypeStruct((N, H), jnp.float32),
  )(parts, parts, x, W_rel, W_root, b_rel.reshape(1, H))


def _final(parts, h1, W_rel, W_root, b_rel, batch3, W1, b1, gamma, beta,
           W2p, b2p):
  def body(p0_ref, p1_ref, h1_ref, wr_ref, wt_ref, br_ref, bt_ref,
           w1_ref, b1_ref, g_ref, be_ref, w2_ref, b2_ref, o_ref, pacc):
    i = pl.program_id(0)

    @pl.when(i == 0)
    def _():
      pacc[...] = jnp.zeros_like(pacc)

    agg = p0_ref[...] + p1_ref[...]
    acc = jnp.dot(agg, wr_ref[...], preferred_element_type=jnp.float32)
    acc += jnp.dot(h1_ref[...], wt_ref[...], preferred_element_type=jnp.float32)
    h2 = jnp.maximum(acc + br_ref[...], 0.0)

    b = bt_ref[0]
    gid = lax.broadcasted_iota(jnp.int32, (G, BLK), 0)
    onehot = (gid == b).astype(jnp.float32)
    pacc[...] += jnp.dot(onehot, h2, preferred_element_type=jnp.float32)

    @pl.when(i == NBLK - 1)
    def _():
      z = jnp.dot(pacc[...], w1_ref[...],
                  preferred_element_type=jnp.float32) + b1_ref[...]
      mu = jnp.mean(z, axis=0, keepdims=True)
      var = jnp.mean((z - mu) * (z - mu), axis=0, keepdims=True)
      zn = (z - mu) * lax.rsqrt(var + 1e-5) * g_ref[...] + be_ref[...]
      zn = jnp.maximum(zn, 0.0)
      logits = jnp.dot(zn, w2_ref[...],
                       preferred_element_type=jnp.float32) + b2_ref[...]
      o_ref[...] = jax.nn.sigmoid(logits)

  H4 = 4 * H
  return pl.pallas_call(
      body,
      grid=(NBLK,),
      in_specs=[
          pl.BlockSpec((BLK, D), lambda i: (i, 0)),
          pl.BlockSpec((BLK, D), lambda i: (i + NBLK, 0)),
          pl.BlockSpec((BLK, H), lambda i: (i, 0)),
          pl.BlockSpec((H, H), lambda i: (0, 0)),
          pl.BlockSpec((H, H), lambda i: (0, 0)),
          pl.BlockSpec((1, H), lambda i: (0, 0)),
          pl.BlockSpec((1, 1, BLK), lambda i: (i, 0, 0)),
          pl.BlockSpec((H, H4), lambda i: (0, 0)),
          pl.BlockSpec((1, H4), lambda i: (0, 0)),
          pl.BlockSpec((1, H4), lambda i: (0, 0)),
          pl.BlockSpec((1, H4), lambda i: (0, 0)),
          pl.BlockSpec((H4, 128), lambda i: (0, 0)),
          pl.BlockSpec((1, 128), lambda i: (0, 0)),
      ],
      out_specs=pl.BlockSpec((G, 128), lambda i: (0, 0)),
      out_shape=jax.ShapeDtypeStruct((G, 128), jnp.float32),
      scratch_shapes=[pltpu.VMEM((G, H), jnp.float32)],
  )(parts, parts, h1, W_rel, W_root, b_rel.reshape(1, H),
    batch3, W1, b1.reshape(1, H4), gamma.reshape(1, H4),
    beta.reshape(1, H4), W2p, b2p)


def kernel(x, edge_index, batch, W_rel0, b_rel0, W_root0, W_rel1, b_rel1,
           W_root1, W1, b1, gamma, beta, W2, b2):
  idx3 = jnp.stack([edge_index[0].reshape(E // K, K),
                    edge_index[1].reshape(E // K, K)], axis=1)
  zeros = jnp.zeros((WCHUNK, D), jnp.float32)
  batch3 = batch.reshape(NBLK, 1, BLK)
  W2p = jnp.pad(W2, ((0, 0), (0, 128 - OUT)))
  b2p = jnp.pad(b2, (0, 128 - OUT)).reshape(1, 128)

  parts0 = _segsum_sc(x, idx3, zeros)
  h1 = _gconv_dense(parts0, x, W_rel0, W_root0, b_rel0)
  parts1 = _segsum_sc(h1, idx3, zeros)
  out = _final(parts1, h1, W_rel1, W_root1, b_rel1, batch3, W1, b1,
               gamma, beta, W2p, b2p)
  return out[:, :OUT]


# speedup vs baseline: 3.2860x; 1.0043x over previous
import functools

import jax
import jax.numpy as jnp
from jax import lax
from jax.experimental import pallas as pl
from jax.experimental.pallas import tpu as pltpu
from jax.experimental.pallas import tpu_sc as plsc

N = 10000
E = 320000
D = 128
H = 128
G = 64
OUT = 1

NC = 2
NS = 16
NW = NC * NS
EPW = E // NW
K = 125
NCHUNK = EPW // K
NRB = 3
NIB = 4
UNROLL = 12
NITER = -(-NCHUNK // UNROLL) * UNROLL
WCHUNK = 80
NWCHUNK = N // WCHUNK
WPT = -(-NWCHUNK // NS)

BLK = 1000
NBLK = N // BLK


def _segsum_sc(table, idx3, zeros):
  mesh = plsc.VectorSubcoreMesh(core_axis_name="c", subcore_axis_name="s")

  @functools.partial(
      pl.kernel,
      out_type=jax.ShapeDtypeStruct((NC * N, D), jnp.float32),
      mesh=mesh,
      scratch_types=[
          pltpu.VMEM((2 * NIB, K), jnp.int32),
          [pltpu.VMEM((K, D), jnp.float32)] * NRB,
          pltpu.VMEM_SHARED((N, D), jnp.float32),
          [pltpu.SemaphoreType.DMA] * NIB,
          [pltpu.SemaphoreType.DMA] * NRB,
          [pltpu.SemaphoreType.DMA] * NRB,
      ],
  )
  def k(table_h, idx_h, zeros_h, out_h,
        idx_v, rows, acc_sh, isem, gsem, ssem):
    c = lax.axis_index("c")
    s = lax.axis_index("s")
    wid = s * NC + c
    base = wid * NCHUNK

    def idx_start(i, b):
      pltpu.async_copy(idx_h.at[base + i], idx_v.at[pl.ds(2 * b, 2)], isem[b])

    def idx_wait(i, b):
      pltpu.make_async_copy(idx_h.at[base + i], idx_v.at[pl.ds(2 * b, 2)],
                            isem[b]).wait()

    def gather_start(q, r):
      pltpu.async_copy(table_h.at[idx_v.at[2 * q]], rows[r], gsem[r])

    def gather_wait(q, r):
      pltpu.make_async_copy(table_h.at[idx_v.at[2 * q]], rows[r],
                            gsem[r]).wait()

    def scatter_start(q, r):
      pltpu.async_copy(rows[r], acc_sh.at[idx_v.at[2 * q + 1]], ssem[r],
                       add=True)

    def scatter_wait(q, r):
      pltpu.make_async_copy(rows[r], acc_sh.at[idx_v.at[2 * q + 1]],
                            ssem[r]).wait()

    def zbody(j, carry):
      m = s + j * NS

      @pl.when(m < NWCHUNK)
      def _():
        pltpu.sync_copy(zeros_h, acc_sh.at[pl.ds(m * WCHUNK, WCHUNK)])

      return carry

    lax.fori_loop(0, WPT, zbody, 0)
    plsc.subcore_barrier()

    for q in range(2):
      idx_start(q, q)
    idx_wait(0, 0)
    gather_start(0, 0)

    def body(t, carry):
      for u in range(UNROLL):
        i = t * UNROLL + u
        r1 = (u + 1) % NRB
        q1 = (u + 1) % NIB
        q2 = (u + 2) % NIB

        @pl.when((i >= 2) & (i < NCHUNK + 2))
        def _():
          scatter_wait(q1, r1)

        @pl.when(i + 2 < NCHUNK)
        def _():
          idx_start(i + 2, q2)

        @pl.when(i + 1 < NCHUNK)
        def _():
          idx_wait(i + 1, q1)
          gather_start(q1, r1)

        @pl.when(i < NCHUNK)
        def _():
          gather_wait(u % NIB, u % NRB)
          scatter_start(u % NIB, u % NRB)
      return carry

    lax.fori_loop(0, NITER // UNROLL, body, 0)

    plsc.subcore_barrier()

    def wbody(j, carry):
      m = s + j * NS

      @pl.when(m < NWCHUNK)
      def _():
        r0 = m * WCHUNK
        pltpu.sync_copy(acc_sh.at[pl.ds(r0, WCHUNK)],
                        out_h.at[pl.ds(c * N + r0, WCHUNK)])

      return carry

    lax.fori_loop(0, WPT, wbody, 0)

  return k(table, idx3, zeros)


def _gconv_dense(parts, x, W_rel, W_root, b_rel):
  def body(p0_ref, p1_ref, x_ref, wr_ref, wt_ref, b_ref, o_ref):
    agg = p0_ref[...] + p1_ref[...]
    acc = jnp.dot(agg, wr_ref[...], preferred_element_type=jnp.float32)
    acc += jnp.dot(x_ref[...], wt_ref[...], preferred_element_type=jnp.float32)
    o_ref[...] = jnp.maximum(acc + b_ref[...], 0.0)

  return pl.pallas_call(
      body,
      grid=(NBLK,),
      in_specs=[
          pl.BlockSpec((BLK, D), lambda i: (i, 0)),
          pl.BlockSpec((BLK, D), lambda i: (i + NBLK, 0)),
          pl.BlockSpec((BLK, D), lambda i: (i, 0)),
          pl.BlockSpec((D, H), lambda i: (0, 0)),
          pl.BlockSpec((D, H), lambda i: (0, 0)),
          pl.BlockSpec((1, H), lambda i: (0, 0)),
      ],
      out_specs=pl.BlockSpec((BLK, H), lambda i: (i, 0)),
      out_shape=jax.ShapeDtypeStruct((N, H), jnp.float32),
  )(parts, parts, x, W_rel, W_root, b_rel.reshape(1, H))


def _final(parts, h1, W_rel, W_root, b_rel, batch3, W1, b1, gamma, beta,
           W2p, b2p):
  def body(p0_ref, p1_ref, h1_ref, wr_ref, wt_ref, br_ref, bt_ref,
           w1_ref, b1_ref, g_ref, be_ref, w2_ref, b2_ref, o_ref, pacc):
    i = pl.program_id(0)

    @pl.when(i == 0)
    def _():
      pacc[...] = jnp.zeros_like(pacc)

    agg = p0_ref[...] + p1_ref[...]
    acc = jnp.dot(agg, wr_ref[...], preferred_element_type=jnp.float32)
    acc += jnp.dot(h1_ref[...], wt_ref[...], preferred_element_type=jnp.float32)
    h2 = jnp.maximum(acc + br_ref[...], 0.0)

    b = bt_ref[0]
    gid = lax.broadcasted_iota(jnp.int32, (G, BLK), 0)
    onehot = (gid == b).astype(jnp.float32)
    pacc[...] += jnp.dot(onehot, h2, preferred_element_type=jnp.float32)

    @pl.when(i == NBLK - 1)
    def _():
      z = jnp.dot(pacc[...], w1_ref[...],
                  preferred_element_type=jnp.float32) + b1_ref[...]
      mu = jnp.mean(z, axis=0, keepdims=True)
      var = jnp.mean((z - mu) * (z - mu), axis=0, keepdims=True)
      zn = (z - mu) * lax.rsqrt(var + 1e-5) * g_ref[...] + be_ref[...]
      zn = jnp.maximum(zn, 0.0)
      logits = jnp.dot(zn, w2_ref[...],
                       preferred_element_type=jnp.float32) + b2_ref[...]
      o_ref[...] = jax.nn.sigmoid(logits)

  H4 = 4 * H
  return pl.pallas_call(
      body,
      grid=(NBLK,),
      in_specs=[
          pl.BlockSpec((BLK, D), lambda i: (i, 0)),
          pl.BlockSpec((BLK, D), lambda i: (i + NBLK, 0)),
          pl.BlockSpec((BLK, H), lambda i: (i, 0)),
          pl.BlockSpec((H, H), lambda i: (0, 0)),
          pl.BlockSpec((H, H), lambda i: (0, 0)),
          pl.BlockSpec((1, H), lambda i: (0, 0)),
          pl.BlockSpec((1, 1, BLK), lambda i: (i, 0, 0)),
          pl.BlockSpec((H, H4), lambda i: (0, 0)),
          pl.BlockSpec((1, H4), lambda i: (0, 0)),
          pl.BlockSpec((1, H4), lambda i: (0, 0)),
          pl.BlockSpec((1, H4), lambda i: (0, 0)),
          pl.BlockSpec((H4, 128), lambda i: (0, 0)),
          pl.BlockSpec((1, 128), lambda i: (0, 0)),
      ],
      out_specs=pl.BlockSpec((G, 128), lambda i: (0, 0)),
      out_shape=jax.ShapeDtypeStruct((G, 128), jnp.float32),
      scratch_shapes=[pltpu.VMEM((G, H), jnp.float32)],
  )(parts, parts, h1, W_rel, W_root, b_rel.reshape(1, H),
    batch3, W1, b1.reshape(1, H4), gamma.reshape(1, H4),
    beta.reshape(1, H4), W2p, b2p)


def kernel(x, edge_index, batch, W_rel0, b_rel0, W_root0, W_rel1, b_rel1,
           W_root1, W1, b1, gamma, beta, W2, b2):
  idx3 = jnp.stack([edge_index[0].reshape(E // K, K),
                    edge_index[1].reshape(E // K, K)], axis=1)
  zeros = jnp.zeros((WCHUNK, D), jnp.float32)
  batch3 = batch.reshape(NBLK, 1, BLK)
  W2p = jnp.pad(W2, ((0, 0), (0, 128 - OUT)))
  b2p = jnp.pad(b2, (0, 128 - OUT)).reshape(1, 128)

  parts0 = _segsum_sc(x, idx3, zeros)
  h1 = _gconv_dense(parts0, x, W_rel0, W_root0, b_rel0)
  parts1 = _segsum_sc(h1, idx3, zeros)
  out = _final(parts1, h1, W_rel1, W_root1, b_rel1, batch3, W1, b1,
               gamma, beta, W2p, b2p)
  return out[:, :OUT]


# speedup vs baseline: 3.3497x; 1.0194x over previous
import functools

import jax
import jax.numpy as jnp
from jax import lax
from jax.experimental import pallas as pl
from jax.experimental.pallas import tpu as pltpu
from jax.experimental.pallas import tpu_sc as plsc

N = 10000
E = 320000
D = 128
H = 128
G = 64
OUT = 1

NC = 2
NS = 16
NW = NC * NS
EPW = E // NW
K = 125
NCHUNK = EPW // K
NRB = 3
NIB = 4
UNROLL = 12
NITER = -(-NCHUNK // UNROLL) * UNROLL
WCHUNK = 80
NWCHUNK = N // WCHUNK
WPT = -(-NWCHUNK // NS)

BLK = 2000
NBLK = N // BLK


def _segsum_sc(table, idx3, zeros):
  mesh = plsc.VectorSubcoreMesh(core_axis_name="c", subcore_axis_name="s")

  @functools.partial(
      pl.kernel,
      out_type=jax.ShapeDtypeStruct((NC * N, D), jnp.float32),
      mesh=mesh,
      scratch_types=[
          pltpu.VMEM((2 * NIB, K), jnp.int32),
          [pltpu.VMEM((K, D), jnp.float32)] * NRB,
          pltpu.VMEM_SHARED((N, D), jnp.float32),
          [pltpu.SemaphoreType.DMA] * NIB,
          [pltpu.SemaphoreType.DMA] * NRB,
          [pltpu.SemaphoreType.DMA] * NRB,
      ],
  )
  def k(table_h, idx_h, zeros_h, out_h,
        idx_v, rows, acc_sh, isem, gsem, ssem):
    c = lax.axis_index("c")
    s = lax.axis_index("s")
    wid = s * NC + c
    base = wid * NCHUNK

    def idx_start(i, b):
      pltpu.async_copy(idx_h.at[base + i], idx_v.at[pl.ds(2 * b, 2)], isem[b])

    def idx_wait(i, b):
      pltpu.make_async_copy(idx_h.at[base + i], idx_v.at[pl.ds(2 * b, 2)],
                            isem[b]).wait()

    def gather_start(q, r):
      pltpu.async_copy(table_h.at[idx_v.at[2 * q]], rows[r], gsem[r])

    def gather_wait(q, r):
      pltpu.make_async_copy(table_h.at[idx_v.at[2 * q]], rows[r],
                            gsem[r]).wait()

    def scatter_start(q, r):
      pltpu.async_copy(rows[r], acc_sh.at[idx_v.at[2 * q + 1]], ssem[r],
                       add=True)

    def scatter_wait(q, r):
      pltpu.make_async_copy(rows[r], acc_sh.at[idx_v.at[2 * q + 1]],
                            ssem[r]).wait()

    def zbody(j, carry):
      m = s + j * NS

      @pl.when(m < NWCHUNK)
      def _():
        pltpu.sync_copy(zeros_h, acc_sh.at[pl.ds(m * WCHUNK, WCHUNK)])

      return carry

    lax.fori_loop(0, WPT, zbody, 0)
    plsc.subcore_barrier()

    for q in range(2):
      idx_start(q, q)
    idx_wait(0, 0)
    gather_start(0, 0)

    def body(t, carry):
      for u in range(UNROLL):
        i = t * UNROLL + u
        r1 = (u + 1) % NRB
        q1 = (u + 1) % NIB
        q2 = (u + 2) % NIB

        @pl.when((i >= 2) & (i < NCHUNK + 2))
        def _():
          scatter_wait(q1, r1)

        @pl.when(i + 2 < NCHUNK)
        def _():
          idx_start(i + 2, q2)

        @pl.when(i + 1 < NCHUNK)
        def _():
          idx_wait(i + 1, q1)
          gather_start(q1, r1)

        @pl.when(i < NCHUNK)
        def _():
          gather_wait(u % NIB, u % NRB)
          scatter_start(u % NIB, u % NRB)
      return carry

    lax.fori_loop(0, NITER // UNROLL, body, 0)

    plsc.subcore_barrier()

    def wbody(j, carry):
      m = s + j * NS

      @pl.when(m < NWCHUNK)
      def _():
        r0 = m * WCHUNK
        pltpu.sync_copy(acc_sh.at[pl.ds(r0, WCHUNK)],
                        out_h.at[pl.ds(c * N + r0, WCHUNK)])

      return carry

    lax.fori_loop(0, WPT, wbody, 0)

  return k(table, idx3, zeros)


def _gconv_dense(parts, x, W_rel, W_root, b_rel):
  def body(p0_ref, p1_ref, x_ref, wr_ref, wt_ref, b_ref, o_ref):
    agg = p0_ref[...] + p1_ref[...]
    acc = jnp.dot(agg, wr_ref[...], preferred_element_type=jnp.float32)
    acc += jnp.dot(x_ref[...], wt_ref[...], preferred_element_type=jnp.float32)
    o_ref[...] = jnp.maximum(acc + b_ref[...], 0.0)

  return pl.pallas_call(
      body,
      grid=(NBLK,),
      in_specs=[
          pl.BlockSpec((BLK, D), lambda i: (i, 0)),
          pl.BlockSpec((BLK, D), lambda i: (i + NBLK, 0)),
          pl.BlockSpec((BLK, D), lambda i: (i, 0)),
          pl.BlockSpec((D, H), lambda i: (0, 0)),
          pl.BlockSpec((D, H), lambda i: (0, 0)),
          pl.BlockSpec((1, H), lambda i: (0, 0)),
      ],
      out_specs=pl.BlockSpec((BLK, H), lambda i: (i, 0)),
      out_shape=jax.ShapeDtypeStruct((N, H), jnp.float32),
  )(parts, parts, x, W_rel, W_root, b_rel.reshape(1, H))


def _final(parts, h1, W_rel, W_root, b_rel, batch3, W1, b1, gamma, beta,
           W2p, b2p):
  def body(p0_ref, p1_ref, h1_ref, wr_ref, wt_ref, br_ref, bt_ref,
           w1_ref, b1_ref, g_ref, be_ref, w2_ref, b2_ref, o_ref, pacc):
    i = pl.program_id(0)

    @pl.when(i == 0)
    def _():
      pacc[...] = jnp.zeros_like(pacc)

    agg = p0_ref[...] + p1_ref[...]
    acc = jnp.dot(agg, wr_ref[...], preferred_element_type=jnp.float32)
    acc += jnp.dot(h1_ref[...], wt_ref[...], preferred_element_type=jnp.float32)
    h2 = jnp.maximum(acc + br_ref[...], 0.0)

    b = bt_ref[0]
    gid = lax.broadcasted_iota(jnp.int32, (G, BLK), 0)
    onehot = (gid == b).astype(jnp.float32)
    pacc[...] += jnp.dot(onehot, h2, preferred_element_type=jnp.float32)

    @pl.when(i == NBLK - 1)
    def _():
      z = jnp.dot(pacc[...], w1_ref[...],
                  preferred_element_type=jnp.float32) + b1_ref[...]
      mu = jnp.mean(z, axis=0, keepdims=True)
      var = jnp.mean((z - mu) * (z - mu), axis=0, keepdims=True)
      zn = (z - mu) * lax.rsqrt(var + 1e-5) * g_ref[...] + be_ref[...]
      zn = jnp.maximum(zn, 0.0)
      logits = jnp.dot(zn, w2_ref[...],
                       preferred_element_type=jnp.float32) + b2_ref[...]
      o_ref[...] = jax.nn.sigmoid(logits)

  H4 = 4 * H
  return pl.pallas_call(
      body,
      grid=(NBLK,),
      in_specs=[
          pl.BlockSpec((BLK, D), lambda i: (i, 0)),
          pl.BlockSpec((BLK, D), lambda i: (i + NBLK, 0)),
          pl.BlockSpec((BLK, H), lambda i: (i, 0)),
          pl.BlockSpec((H, H), lambda i: (0, 0)),
          pl.BlockSpec((H, H), lambda i: (0, 0)),
          pl.BlockSpec((1, H), lambda i: (0, 0)),
          pl.BlockSpec((1, 1, BLK), lambda i: (i, 0, 0)),
          pl.BlockSpec((H, H4), lambda i: (0, 0)),
          pl.BlockSpec((1, H4), lambda i: (0, 0)),
          pl.BlockSpec((1, H4), lambda i: (0, 0)),
          pl.BlockSpec((1, H4), lambda i: (0, 0)),
          pl.BlockSpec((H4, 128), lambda i: (0, 0)),
          pl.BlockSpec((1, 128), lambda i: (0, 0)),
      ],
      out_specs=pl.BlockSpec((G, 128), lambda i: (0, 0)),
      out_shape=jax.ShapeDtypeStruct((G, 128), jnp.float32),
      scratch_shapes=[pltpu.VMEM((G, H), jnp.float32)],
  )(parts, parts, h1, W_rel, W_root, b_rel.reshape(1, H),
    batch3, W1, b1.reshape(1, H4), gamma.reshape(1, H4),
    beta.reshape(1, H4), W2p, b2p)


def kernel(x, edge_index, batch, W_rel0, b_rel0, W_root0, W_rel1, b_rel1,
           W_root1, W1, b1, gamma, beta, W2, b2):
  idx3 = jnp.stack([edge_index[0].reshape(E // K, K),
                    edge_index[1].reshape(E // K, K)], axis=1)
  zeros = jnp.zeros((WCHUNK, D), jnp.float32)
  batch3 = batch.reshape(NBLK, 1, BLK)
  W2p = jnp.pad(W2, ((0, 0), (0, 128 - OUT)))
  b2p = jnp.pad(b2, (0, 128 - OUT)).reshape(1, 128)

  parts0 = _segsum_sc(x, idx3, zeros)
  h1 = _gconv_dense(parts0, x, W_rel0, W_root0, b_rel0)
  parts1 = _segsum_sc(h1, idx3, zeros)
  out = _final(parts1, h1, W_rel1, W_root1, b_rel1, batch3, W1, b1,
               gamma, beta, W2p, b2p)
  return out[:, :OUT]


# speedup vs baseline: 3.3809x; 1.0093x over previous
import functools

import jax
import jax.numpy as jnp
from jax import lax
from jax.experimental import pallas as pl
from jax.experimental.pallas import tpu as pltpu
from jax.experimental.pallas import tpu_sc as plsc

N = 10000
E = 320000
D = 128
H = 128
G = 64
OUT = 1

NC = 2
NS = 16
NW = NC * NS
EPW = E // NW
K = 125
NCHUNK = EPW // K
NRB = 3
NIB = 4
UNROLL = 12
NITER = -(-NCHUNK // UNROLL) * UNROLL
WCHUNK = 80
NWCHUNK = N // WCHUNK
WPT = -(-NWCHUNK // NS)

BLK = 2000
NBLK = N // BLK


def _segsum_sc(table, idx3, zeros):
  mesh = plsc.VectorSubcoreMesh(core_axis_name="c", subcore_axis_name="s")

  @functools.partial(
      pl.kernel,
      out_type=jax.ShapeDtypeStruct((NC * N, D), jnp.float32),
      mesh=mesh,
      scratch_types=[
          pltpu.VMEM((2 * NIB, K), jnp.int32),
          [pltpu.VMEM((K, D), jnp.float32)] * NRB,
          pltpu.VMEM_SHARED((N, D), jnp.float32),
          [pltpu.SemaphoreType.DMA] * NIB,
          [pltpu.SemaphoreType.DMA] * NRB,
          [pltpu.SemaphoreType.DMA] * NRB,
      ],
  )
  def k(table_h, idx_h, zeros_h, out_h,
        idx_v, rows, acc_sh, isem, gsem, ssem):
    c = lax.axis_index("c")
    s = lax.axis_index("s")
    wid = s * NC + c
    base = wid * NCHUNK

    def idx_start(i, b):
      pltpu.async_copy(idx_h.at[base + i], idx_v.at[pl.ds(2 * b, 2)], isem[b])

    def idx_wait(i, b):
      pltpu.make_async_copy(idx_h.at[base + i], idx_v.at[pl.ds(2 * b, 2)],
                            isem[b]).wait()

    def gather_start(q, r):
      pltpu.async_copy(table_h.at[idx_v.at[2 * q]], rows[r], gsem[r])

    def gather_wait(q, r):
      pltpu.make_async_copy(table_h.at[idx_v.at[2 * q]], rows[r],
                            gsem[r]).wait()

    def scatter_start(q, r):
      pltpu.async_copy(rows[r], acc_sh.at[idx_v.at[2 * q + 1]], ssem[r],
                       add=True)

    def scatter_wait(q, r):
      pltpu.make_async_copy(rows[r], acc_sh.at[idx_v.at[2 * q + 1]],
                            ssem[r]).wait()

    def zbody(j, carry):
      m = s + j * NS

      @pl.when(m < NWCHUNK)
      def _():
        pltpu.sync_copy(zeros_h, acc_sh.at[pl.ds(m * WCHUNK, WCHUNK)])

      return carry

    for q in range(2):
      idx_start(q, q)
    lax.fori_loop(0, WPT, zbody, 0)
    idx_wait(0, 0)
    gather_start(0, 0)
    plsc.subcore_barrier()

    def body(t, carry):
      for u in range(UNROLL):
        i = t * UNROLL + u
        r1 = (u + 1) % NRB
        q1 = (u + 1) % NIB
        q2 = (u + 2) % NIB

        @pl.when((i >= 2) & (i < NCHUNK + 2))
        def _():
          scatter_wait(q1, r1)

        @pl.when(i + 2 < NCHUNK)
        def _():
          idx_start(i + 2, q2)

        @pl.when(i + 1 < NCHUNK)
        def _():
          idx_wait(i + 1, q1)
          gather_start(q1, r1)

        @pl.when(i < NCHUNK)
        def _():
          gather_wait(u % NIB, u % NRB)
          scatter_start(u % NIB, u % NRB)
      return carry

    lax.fori_loop(0, NITER // UNROLL, body, 0)

    plsc.subcore_barrier()

    def wbody(j, carry):
      m = s + j * NS

      @pl.when(m < NWCHUNK)
      def _():
        r0 = m * WCHUNK
        pltpu.sync_copy(acc_sh.at[pl.ds(r0, WCHUNK)],
                        out_h.at[pl.ds(c * N + r0, WCHUNK)])

      return carry

    lax.fori_loop(0, WPT, wbody, 0)

  return k(table, idx3, zeros)


def _gconv_dense(parts, x, W_rel, W_root, b_rel):
  def body(p0_ref, p1_ref, x_ref, wr_ref, wt_ref, b_ref, o_ref):
    agg = p0_ref[...] + p1_ref[...]
    acc = jnp.dot(agg, wr_ref[...], preferred_element_type=jnp.float32)
    acc += jnp.dot(x_ref[...], wt_ref[...], preferred_element_type=jnp.float32)
    o_ref[...] = jnp.maximum(acc + b_ref[...], 0.0)

  return pl.pallas_call(
      body,
      grid=(NBLK,),
      in_specs=[
          pl.BlockSpec((BLK, D), lambda i: (i, 0)),
          pl.BlockSpec((BLK, D), lambda i: (i + NBLK, 0)),
          pl.BlockSpec((BLK, D), lambda i: (i, 0)),
          pl.BlockSpec((D, H), lambda i: (0, 0)),
          pl.BlockSpec((D, H), lambda i: (0, 0)),
          pl.BlockSpec((1, H), lambda i: (0, 0)),
      ],
      out_specs=pl.BlockSpec((BLK, H), lambda i: (i, 0)),
      out_shape=jax.ShapeDtypeStruct((N, H), jnp.float32),
  )(parts, parts, x, W_rel, W_root, b_rel.reshape(1, H))


def _final(parts, h1, W_rel, W_root, b_rel, batch3, W1, b1, gamma, beta,
           W2p, b2p):
  def body(p0_ref, p1_ref, h1_ref, wr_ref, wt_ref, br_ref, bt_ref,
           w1_ref, b1_ref, g_ref, be_ref, w2_ref, b2_ref, o_ref, pacc):
    i = pl.program_id(0)

    @pl.when(i == 0)
    def _():
      pacc[...] = jnp.zeros_like(pacc)

    agg = p0_ref[...] + p1_ref[...]
    acc = jnp.dot(agg, wr_ref[...], preferred_element_type=jnp.float32)
    acc += jnp.dot(h1_ref[...], wt_ref[...], preferred_element_type=jnp.float32)
    h2 = jnp.maximum(acc + br_ref[...], 0.0)

    b = bt_ref[0]
    gid = lax.broadcasted_iota(jnp.int32, (G, BLK), 0)
    onehot = (gid == b).astype(jnp.float32)
    pacc[...] += jnp.dot(onehot, h2, preferred_element_type=jnp.float32)

    @pl.when(i == NBLK - 1)
    def _():
      z = jnp.dot(pacc[...], w1_ref[...],
                  preferred_element_type=jnp.float32) + b1_ref[...]
      mu = jnp.mean(z, axis=0, keepdims=True)
      var = jnp.mean((z - mu) * (z - mu), axis=0, keepdims=True)
      zn = (z - mu) * lax.rsqrt(var + 1e-5) * g_ref[...] + be_ref[...]
      zn = jnp.maximum(zn, 0.0)
      logits = jnp.dot(zn, w2_ref[...],
                       preferred_element_type=jnp.float32) + b2_ref[...]
      o_ref[...] = jax.nn.sigmoid(logits)

  H4 = 4 * H
  return pl.pallas_call(
      body,
      grid=(NBLK,),
      in_specs=[
          pl.BlockSpec((BLK, D), lambda i: (i, 0)),
          pl.BlockSpec((BLK, D), lambda i: (i + NBLK, 0)),
          pl.BlockSpec((BLK, H), lambda i: (i, 0)),
          pl.BlockSpec((H, H), lambda i: (0, 0)),
          pl.BlockSpec((H, H), lambda i: (0, 0)),
          pl.BlockSpec((1, H), lambda i: (0, 0)),
          pl.BlockSpec((1, 1, BLK), lambda i: (i, 0, 0)),
          pl.BlockSpec((H, H4), lambda i: (0, 0)),
          pl.BlockSpec((1, H4), lambda i: (0, 0)),
          pl.BlockSpec((1, H4), lambda i: (0, 0)),
          pl.BlockSpec((1, H4), lambda i: (0, 0)),
          pl.BlockSpec((H4, 128), lambda i: (0, 0)),
          pl.BlockSpec((1, 128), lambda i: (0, 0)),
      ],
      out_specs=pl.BlockSpec((G, 128), lambda i: (0, 0)),
      out_shape=jax.ShapeDtypeStruct((G, 128), jnp.float32),
      scratch_shapes=[pltpu.VMEM((G, H), jnp.float32)],
  )(parts, parts, h1, W_rel, W_root, b_rel.reshape(1, H),
    batch3, W1, b1.reshape(1, H4), gamma.reshape(1, H4),
    beta.reshape(1, H4), W2p, b2p)


def kernel(x, edge_index, batch, W_rel0, b_rel0, W_root0, W_rel1, b_rel1,
           W_root1, W1, b1, gamma, beta, W2, b2):
  idx3 = jnp.stack([edge_index[0].reshape(E // K, K),
                    edge_index[1].reshape(E // K, K)], axis=1)
  zeros = jnp.zeros((WCHUNK, D), jnp.float32)
  batch3 = batch.reshape(NBLK, 1, BLK)
  W2p = jnp.pad(W2, ((0, 0), (0, 128 - OUT)))
  b2p = jnp.pad(b2, (0, 128 - OUT)).reshape(1, 128)

  parts0 = _segsum_sc(x, idx3, zeros)
  h1 = _gconv_dense(parts0, x, W_rel0, W_root0, b_rel0)
  parts1 = _segsum_sc(h1, idx3, zeros)
  out = _final(parts1, h1, W_rel1, W_root1, b_rel1, batch3, W1, b1,
               gamma, beta, W2p, b2p)
  return out[:, :OUT]
